# baseline (speedup 1.0000x reference)
import functools

import jax
import jax.numpy as jnp
from jax import lax
from jax.experimental import pallas as pl
from jax.experimental.pallas import tpu as pltpu
from jax.experimental.pallas import tpu_sc as plsc

N_NODES = 10000
N_EDGES = 320000
NUM_GRAPHS = 64
DIN = 16
HID = 128

NC, NS = 2, 16
NW = NC * NS
NPAD = 10240
ROWS_PER_TILE = NPAD // NS
CHUNK = 64
NCH = 160
NCHH = 80
EPW = NCH * CHUNK
E_PAD = NW * EPW
DUMMY = NPAD - 1

_mesh = plsc.VectorSubcoreMesh(core_axis_name="c", subcore_axis_name="s")


def _zero_fill(ref, nrows, ncols):
    zeros16 = jnp.zeros((16,), jnp.float32)

    def body(r, _):
        for c in range(ncols // 16):
            ref[r, c * 16:(c + 1) * 16] = zeros16
        return 0

    lax.fori_loop(0, nrows, body, 0)


def _sc_deg(dstp):
    def body(dst_hbm, out_hbm, idx_v, ones_v, zer_v, shared):
        cid = lax.axis_index("c")
        sid = lax.axis_index("s")
        wid = cid * NS + sid
        for c in range(CHUNK // 16):
            ones_v[c * 16:(c + 1) * 16] = jnp.full((16,), 1.0, jnp.float32)
            for r in range(ROWS_PER_TILE // CHUNK):
                zer_v[r * CHUNK + c * 16:r * CHUNK + (c + 1) * 16] = (
                    jnp.zeros((16,), jnp.float32))
        pltpu.sync_copy(zer_v, shared.at[pl.ds(sid * ROWS_PER_TILE, ROWS_PER_TILE)])
        plsc.subcore_barrier()
        pltpu.sync_copy(dst_hbm.at[wid], idx_v)

        def step(j, _):
            pltpu.sync_copy(ones_v, shared.at[idx_v.at[j]], add=True)
            return 0

        lax.fori_loop(0, NCH, step, 0)
        plsc.subcore_barrier()
        pltpu.sync_copy(
            shared.at[pl.ds(sid * ROWS_PER_TILE, ROWS_PER_TILE)],
            out_hbm.at[cid, pl.ds(sid * ROWS_PER_TILE, ROWS_PER_TILE)])

    return pl.kernel(
        body,
        out_type=jax.ShapeDtypeStruct((NC, NPAD), jnp.float32),
        mesh=_mesh,
        scratch_types=[
            pltpu.VMEM((NCH, CHUNK), jnp.int32),
            pltpu.VMEM((CHUNK,), jnp.float32),
            pltpu.VMEM((ROWS_PER_TILE,), jnp.float32),
            pltpu.VMEM_SHARED((NPAD,), jnp.float32),
        ],
    )(dstp)


def _sc_agg(table, srcp, dstp, D, untiled):
    def body(tab_hbm, src_hbm, dst_hbm, out_hbm, sidx, didx, rows, rows1,
             shared, sem0, sem1, ssem0, ssem1):
        cid = lax.axis_index("c")
        sid = lax.axis_index("s")
        wid = cid * NS + sid
        _zero_fill(rows, CHUNK, D)
        for r in range(ROWS_PER_TILE // CHUNK):
            pltpu.sync_copy(
                rows, shared.at[pl.ds(sid * ROWS_PER_TILE + r * CHUNK, CHUNK)])
        plsc.subcore_barrier()

        def step(g, _):
            @pl.when(g > 0)
            def _():
                pltpu.make_async_copy(
                    rows1, shared.at[didx.at[2 * g - 1]], ssem1).wait()

            pltpu.async_copy(tab_hbm.at[sidx.at[2 * g + 1]], rows1, sem1)
            pltpu.make_async_copy(tab_hbm.at[sidx.at[2 * g]], rows, sem0).wait()
            pltpu.async_copy(rows, shared.at[didx.at[2 * g]], ssem0, add=True)
            pltpu.make_async_copy(
                tab_hbm.at[sidx.at[2 * g + 1]], rows1, sem1).wait()
            pltpu.async_copy(
                rows1, shared.at[didx.at[2 * g + 1]], ssem1, add=True)
            pltpu.make_async_copy(
                rows, shared.at[didx.at[2 * g]], ssem0).wait()
            nxt = jnp.minimum(2 * g + 2, NCHH - 1)
            pltpu.async_copy(tab_hbm.at[sidx.at[nxt]], rows, sem0)
            return 0

        for half in range(NCH // NCHH):
            pltpu.sync_copy(src_hbm.at[wid, pl.ds(half * NCHH, NCHH)], sidx)
            pltpu.sync_copy(dst_hbm.at[wid, pl.ds(half * NCHH, NCHH)], didx)
            pltpu.async_copy(tab_hbm.at[sidx.at[0]], rows, sem0)
            lax.fori_loop(0, NCHH // 2, step, 0)
            pltpu.make_async_copy(
                rows1, shared.at[didx.at[NCHH - 1]], ssem1).wait()
            pltpu.make_async_copy(tab_hbm.at[sidx.at[NCHH - 1]], rows, sem0).wait()
        plsc.subcore_barrier()
        pltpu.sync_copy(
            shared.at[pl.ds(sid * ROWS_PER_TILE, ROWS_PER_TILE)],
            out_hbm.at[cid, pl.ds(sid * ROWS_PER_TILE, ROWS_PER_TILE)])

    return pl.kernel(
        body,
        out_type=jax.ShapeDtypeStruct((NC, NPAD, D), jnp.float32),
        mesh=_mesh,
        compiler_params=(pltpu.CompilerParams(use_tc_tiling_on_sc=False)
                         if untiled else None),
        scratch_types=[
            pltpu.VMEM((NCHH, CHUNK), jnp.int32),
            pltpu.VMEM((NCHH, CHUNK), jnp.int32),
            pltpu.VMEM((CHUNK, D), jnp.float32),
            pltpu.VMEM((CHUNK, D), jnp.float32),
            pltpu.VMEM_SHARED((NPAD, D), jnp.float32),
            pltpu.SemaphoreType.DMA,
            pltpu.SemaphoreType.DMA,
            pltpu.SemaphoreType.DMA,
            pltpu.SemaphoreType.DMA,
        ],
    )(table, srcp, dstp)


def _tc_prep_body(degp_ref, x_ref, w1_ref, z1_ref, dinv_ref):
    deg = degp_ref[0, :] + degp_ref[1, :] + 1.0
    dinv = 1.0 / jnp.sqrt(deg)
    dinv_ref[...] = dinv
    xw = jnp.dot(x_ref[...], w1_ref[...], preferred_element_type=jnp.float32)
    z1_ref[...] = xw * dinv[:, None]


def _tc_prep(degp, x_pad, W1p):
    return pl.pallas_call(
        _tc_prep_body,
        out_shape=(jax.ShapeDtypeStruct((NPAD, HID), jnp.float32),
                   jax.ShapeDtypeStruct((NPAD,), jnp.float32)),
    )(degp, x_pad, W1p)


def _tc_mid_body(s1p_ref, z1_ref, dinv_ref, b1_ref, w2_ref, z2_ref):
    t = s1p_ref[0] + s1p_ref[1] + z1_ref[...]
    dinv = dinv_ref[...]
    h1 = jnp.maximum(dinv[:, None] * t + b1_ref[...][None, :], 0.0)
    z2 = dinv[:, None] * jnp.dot(h1, w2_ref[...],
                                 preferred_element_type=jnp.float32)
    z2_ref[...] = z2


def _tc_mid(s1p, z1, dinv, b1, W2):
    blk = 2048
    grid = NPAD // blk
    return pl.pallas_call(
        _tc_mid_body,
        grid=(grid,),
        in_specs=[
            pl.BlockSpec((NC, blk, HID), lambda i: (0, i, 0)),
            pl.BlockSpec((blk, HID), lambda i: (i, 0)),
            pl.BlockSpec((blk,), lambda i: (i,)),
            pl.BlockSpec((HID,), lambda i: (0,)),
            pl.BlockSpec((HID, HID), lambda i: (0, 0)),
        ],
        out_specs=pl.BlockSpec((blk, HID), lambda i: (i, 0)),
        out_shape=jax.ShapeDtypeStruct((NPAD, HID), jnp.float32),
    )(s1p, z1, dinv, b1, W2)


GPAD = 128


def _tc_final_body(s2p_ref, z2_ref, dinv_ref, b2_ref, wlin_ref, blin_ref,
                   bat_ref, out_ref, sum_ref, cnt_ref):
    i = pl.program_id(0)
    t = s2p_ref[0] + s2p_ref[1] + z2_ref[...]
    h2 = jnp.maximum(dinv_ref[...][:, None] * t + b2_ref[...][None, :], 0.0)
    nblk = h2.shape[0]
    gids = lax.broadcasted_iota(jnp.int32, (NUM_GRAPHS, nblk), 0).astype(jnp.float32)
    match = (bat_ref[...][None, :] == gids).astype(jnp.float32)
    sums = jnp.dot(match, h2, preferred_element_type=jnp.float32,
                   precision=lax.Precision.HIGHEST)
    cnts = jnp.sum(match, axis=1)

    @pl.when(i == 0)
    def _():
        sum_ref[...] = sums
        cnt_ref[...] = cnts

    @pl.when(i > 0)
    def _():
        sum_ref[...] = sum_ref[...] + sums
        cnt_ref[...] = cnt_ref[...] + cnts

    @pl.when(i == pl.num_programs(0) - 1)
    def _():
        g = sum_ref[...] / jnp.maximum(cnt_ref[...], 1.0)[:, None]
        out_ref[...] = jnp.dot(
            g, wlin_ref[...], preferred_element_type=jnp.float32)[:, 0] + blin_ref[0]


def _tc_final(s2p, z2, dinv, b2, Wlin, blin, batf):
    blk = 2048
    grid = NPAD // blk
    return pl.pallas_call(
        _tc_final_body,
        grid=(grid,),
        in_specs=[
            pl.BlockSpec((NC, blk, HID), lambda i: (0, i, 0)),
            pl.BlockSpec((blk, HID), lambda i: (i, 0)),
            pl.BlockSpec((blk,), lambda i: (i,)),
            pl.BlockSpec((HID,), lambda i: (0,)),
            pl.BlockSpec((HID, 1), lambda i: (0, 0)),
            pl.BlockSpec((1,), lambda i: (0,)),
            pl.BlockSpec((blk,), lambda i: (i,)),
        ],
        out_specs=pl.BlockSpec((NUM_GRAPHS,), lambda i: (0,)),
        out_shape=jax.ShapeDtypeStruct((NUM_GRAPHS,), jnp.float32),
        scratch_shapes=[pltpu.VMEM((NUM_GRAPHS, HID), jnp.float32),
                        pltpu.VMEM((NUM_GRAPHS,), jnp.float32)],
    )(s2p, z2, dinv, b2, Wlin, blin, batf)


def kernel(x, edge_index, batch, W1, b1, W2, b2, Wlin, blin):
    src = edge_index[0].astype(jnp.int32)
    dst = edge_index[1].astype(jnp.int32)
    epad = N_NODES + jnp.arange(E_PAD - N_EDGES, dtype=jnp.int32) % (NPAD - N_NODES)
    srcp = jnp.concatenate([src, epad]).reshape(NW, NCH, CHUNK)
    dstp = jnp.concatenate([dst, epad]).reshape(NW, NCH, CHUNK)
    batf = jnp.concatenate([
        batch.astype(jnp.float32),
        jnp.full((NPAD - N_NODES,), float(NUM_GRAPHS), jnp.float32)])
    x_pad = jnp.zeros((NPAD, DIN), jnp.float32).at[:N_NODES, :x.shape[1]].set(x)
    W1p = jnp.zeros((DIN, HID), jnp.float32).at[:W1.shape[0]].set(W1)

    degp = _sc_deg(dstp)
    z1, dinv = _tc_prep(degp, x_pad, W1p)
    s1p = _sc_agg(z1, srcp, dstp, D=HID, untiled=False)
    z2 = _tc_mid(s1p, z1, dinv, b1, W2)
    s2p = _sc_agg(z2, srcp, dstp, D=HID, untiled=False)
    return _tc_final(s2p, z2, dinv, b2, Wlin, blin, batf)


# speedup vs baseline: 1.3214x; 1.3214x over previous
import functools

import jax
import jax.numpy as jnp
from jax import lax
from jax.experimental import pallas as pl
from jax.experimental.pallas import tpu as pltpu
from jax.experimental.pallas import tpu_sc as plsc

N_NODES = 10000
N_EDGES = 320000
NUM_GRAPHS = 64
DIN = 16
HID = 128

NC, NS = 2, 16
NW = NC * NS
NPAD = 10240
ROWS_PER_TILE = NPAD // NS
CHUNK = 128
NCH = 80
NCHH = 16
EPW = NCH * CHUNK
E_PAD = NW * EPW
DUMMY = NPAD - 1

_mesh = plsc.VectorSubcoreMesh(core_axis_name="c", subcore_axis_name="s")


def _zero_fill(ref, nrows, ncols):
    zeros16 = jnp.zeros((16,), jnp.float32)

    def body(r, _):
        for c in range(ncols // 16):
            ref[r, c * 16:(c + 1) * 16] = zeros16
        return 0

    lax.fori_loop(0, nrows, body, 0)


def _sc_deg(dstp):
    def body(dst_hbm, out_hbm, idx_v, ones_v, zer_v, shared):
        cid = lax.axis_index("c")
        sid = lax.axis_index("s")
        wid = cid * NS + sid
        for c in range(CHUNK // 16):
            ones_v[c * 16:(c + 1) * 16] = jnp.full((16,), 1.0, jnp.float32)
            for r in range(ROWS_PER_TILE // CHUNK):
                zer_v[r * CHUNK + c * 16:r * CHUNK + (c + 1) * 16] = (
                    jnp.zeros((16,), jnp.float32))
        pltpu.sync_copy(zer_v, shared.at[pl.ds(sid * ROWS_PER_TILE, ROWS_PER_TILE)])
        plsc.subcore_barrier()
        pltpu.sync_copy(dst_hbm.at[wid], idx_v)

        def step(j, _):
            pltpu.sync_copy(ones_v, shared.at[idx_v.at[j]], add=True)
            return 0

        lax.fori_loop(0, NCH, step, 0)
        plsc.subcore_barrier()
        pltpu.sync_copy(
            shared.at[pl.ds(sid * ROWS_PER_TILE, ROWS_PER_TILE)],
            out_hbm.at[cid, pl.ds(sid * ROWS_PER_TILE, ROWS_PER_TILE)])

    return pl.kernel(
        body,
        out_type=jax.ShapeDtypeStruct((NC, NPAD), jnp.float32),
        mesh=_mesh,
        scratch_types=[
            pltpu.VMEM((NCH, CHUNK), jnp.int32),
            pltpu.VMEM((CHUNK,), jnp.float32),
            pltpu.VMEM((ROWS_PER_TILE,), jnp.float32),
            pltpu.VMEM_SHARED((NPAD,), jnp.float32),
        ],
    )(dstp)


def _sc_agg(table, srcp, dstp, D, untiled):
    def body(tab_hbm, src_hbm, dst_hbm, out_hbm, sidx, didx, rows, rows1,
             shared, sem0, sem1):
        cid = lax.axis_index("c")
        sid = lax.axis_index("s")
        wid = cid * NS + sid
        _zero_fill(rows, CHUNK, D)
        for r in range(ROWS_PER_TILE // CHUNK):
            pltpu.sync_copy(
                rows, shared.at[pl.ds(sid * ROWS_PER_TILE + r * CHUNK, CHUNK)])
        plsc.subcore_barrier()

        def step(g, _):
            pltpu.async_copy(tab_hbm.at[sidx.at[2 * g + 1]], rows1, sem1)
            pltpu.make_async_copy(tab_hbm.at[sidx.at[2 * g]], rows, sem0).wait()
            pltpu.sync_copy(rows, shared.at[didx.at[2 * g]], add=True)
            nxt = jnp.minimum(2 * g + 2, NCHH - 1)
            pltpu.async_copy(tab_hbm.at[sidx.at[nxt]], rows, sem0)
            pltpu.make_async_copy(
                tab_hbm.at[sidx.at[2 * g + 1]], rows1, sem1).wait()
            pltpu.sync_copy(rows1, shared.at[didx.at[2 * g + 1]], add=True)
            return 0

        for half in range(NCH // NCHH):
            pltpu.sync_copy(src_hbm.at[wid, pl.ds(half * NCHH, NCHH)], sidx)
            pltpu.sync_copy(dst_hbm.at[wid, pl.ds(half * NCHH, NCHH)], didx)
            pltpu.async_copy(tab_hbm.at[sidx.at[0]], rows, sem0)
            lax.fori_loop(0, NCHH // 2, step, 0)
            pltpu.make_async_copy(tab_hbm.at[sidx.at[NCHH - 1]], rows, sem0).wait()
        plsc.subcore_barrier()
        pltpu.sync_copy(
            shared.at[pl.ds(sid * ROWS_PER_TILE, ROWS_PER_TILE)],
            out_hbm.at[cid, pl.ds(sid * ROWS_PER_TILE, ROWS_PER_TILE)])

    return pl.kernel(
        body,
        out_type=jax.ShapeDtypeStruct((NC, NPAD, D), jnp.float32),
        mesh=_mesh,
        compiler_params=(pltpu.CompilerParams(use_tc_tiling_on_sc=False)
                         if untiled else None),
        scratch_types=[
            pltpu.VMEM((NCHH, CHUNK), jnp.int32),
            pltpu.VMEM((NCHH, CHUNK), jnp.int32),
            pltpu.VMEM((CHUNK, D), jnp.float32),
            pltpu.VMEM((CHUNK, D), jnp.float32),
            pltpu.VMEM_SHARED((NPAD, D), jnp.float32),
            pltpu.SemaphoreType.DMA,
            pltpu.SemaphoreType.DMA,
        ],
    )(table, srcp, dstp)


def _tc_prep_body(degp_ref, x_ref, w1_ref, z1_ref, dinv_ref):
    deg = degp_ref[0, :] + degp_ref[1, :] + 1.0
    dinv = 1.0 / jnp.sqrt(deg)
    dinv_ref[...] = dinv
    xw = jnp.dot(x_ref[...], w1_ref[...], preferred_element_type=jnp.float32)
    z1_ref[...] = xw * dinv[:, None]


def _tc_prep(degp, x_pad, W1p):
    return pl.pallas_call(
        _tc_prep_body,
        out_shape=(jax.ShapeDtypeStruct((NPAD, HID), jnp.float32),
                   jax.ShapeDtypeStruct((NPAD,), jnp.float32)),
    )(degp, x_pad, W1p)


def _tc_mid_body(s1p_ref, z1_ref, dinv_ref, b1_ref, w2_ref, z2_ref):
    t = s1p_ref[0] + s1p_ref[1] + z1_ref[...]
    dinv = dinv_ref[...]
    h1 = jnp.maximum(dinv[:, None] * t + b1_ref[...][None, :], 0.0)
    z2 = dinv[:, None] * jnp.dot(h1, w2_ref[...],
                                 preferred_element_type=jnp.float32)
    z2_ref[...] = z2


def _tc_mid(s1p, z1, dinv, b1, W2):
    blk = 2048
    grid = NPAD // blk
    return pl.pallas_call(
        _tc_mid_body,
        grid=(grid,),
        in_specs=[
            pl.BlockSpec((NC, blk, HID), lambda i: (0, i, 0)),
            pl.BlockSpec((blk, HID), lambda i: (i, 0)),
            pl.BlockSpec((blk,), lambda i: (i,)),
            pl.BlockSpec((HID,), lambda i: (0,)),
            pl.BlockSpec((HID, HID), lambda i: (0, 0)),
        ],
        out_specs=pl.BlockSpec((blk, HID), lambda i: (i, 0)),
        out_shape=jax.ShapeDtypeStruct((NPAD, HID), jnp.float32),
    )(s1p, z1, dinv, b1, W2)


GPAD = 128


def _tc_final_body(s2p_ref, z2_ref, dinv_ref, b2_ref, wlin_ref, blin_ref,
                   bat_ref, out_ref, sum_ref, cnt_ref):
    i = pl.program_id(0)
    t = s2p_ref[0] + s2p_ref[1] + z2_ref[...]
    h2 = jnp.maximum(dinv_ref[...][:, None] * t + b2_ref[...][None, :], 0.0)
    nblk = h2.shape[0]
    gids = lax.broadcasted_iota(jnp.int32, (NUM_GRAPHS, nblk), 0).astype(jnp.float32)
    match = (bat_ref[...][None, :] == gids).astype(jnp.float32)
    sums = jnp.dot(match, h2, preferred_element_type=jnp.float32,
                   precision=lax.Precision.HIGHEST)
    cnts = jnp.sum(match, axis=1)

    @pl.when(i == 0)
    def _():
        sum_ref[...] = sums
        cnt_ref[...] = cnts

    @pl.when(i > 0)
    def _():
        sum_ref[...] = sum_ref[...] + sums
        cnt_ref[...] = cnt_ref[...] + cnts

    @pl.when(i == pl.num_programs(0) - 1)
    def _():
        g = sum_ref[...] / jnp.maximum(cnt_ref[...], 1.0)[:, None]
        out_ref[...] = jnp.dot(
            g, wlin_ref[...], preferred_element_type=jnp.float32)[:, 0] + blin_ref[0]


def _tc_final(s2p, z2, dinv, b2, Wlin, blin, batf):
    blk = 2048
    grid = NPAD // blk
    return pl.pallas_call(
        _tc_final_body,
        grid=(grid,),
        in_specs=[
            pl.BlockSpec((NC, blk, HID), lambda i: (0, i, 0)),
            pl.BlockSpec((blk, HID), lambda i: (i, 0)),
            pl.BlockSpec((blk,), lambda i: (i,)),
            pl.BlockSpec((HID,), lambda i: (0,)),
            pl.BlockSpec((HID, 1), lambda i: (0, 0)),
            pl.BlockSpec((1,), lambda i: (0,)),
            pl.BlockSpec((blk,), lambda i: (i,)),
        ],
        out_specs=pl.BlockSpec((NUM_GRAPHS,), lambda i: (0,)),
        out_shape=jax.ShapeDtypeStruct((NUM_GRAPHS,), jnp.float32),
        scratch_shapes=[pltpu.VMEM((NUM_GRAPHS, HID), jnp.float32),
                        pltpu.VMEM((NUM_GRAPHS,), jnp.float32)],
    )(s2p, z2, dinv, b2, Wlin, blin, batf)


def kernel(x, edge_index, batch, W1, b1, W2, b2, Wlin, blin):
    src = edge_index[0].astype(jnp.int32)
    dst = edge_index[1].astype(jnp.int32)
    epad = N_NODES + jnp.arange(E_PAD - N_EDGES, dtype=jnp.int32) % (NPAD - N_NODES)
    srcp = jnp.concatenate([src, epad]).reshape(NW, NCH, CHUNK)
    dstp = jnp.concatenate([dst, epad]).reshape(NW, NCH, CHUNK)
    batf = jnp.concatenate([
        batch.astype(jnp.float32),
        jnp.full((NPAD - N_NODES,), float(NUM_GRAPHS), jnp.float32)])
    x_pad = jnp.zeros((NPAD, DIN), jnp.float32).at[:N_NODES, :x.shape[1]].set(x)
    W1p = jnp.zeros((DIN, HID), jnp.float32).at[:W1.shape[0]].set(W1)

    degp = _sc_deg(dstp)
    z1, dinv = _tc_prep(degp, x_pad, W1p)
    s1p = _sc_agg(z1, srcp, dstp, D=HID, untiled=False)
    z2 = _tc_mid(s1p, z1, dinv, b1, W2)
    s2p = _sc_agg(z2, srcp, dstp, D=HID, untiled=False)
    return _tc_final(s2p, z2, dinv, b2, Wlin, blin, batf)
